# baseline (speedup 1.0000x reference)
import jax
import jax.numpy as jnp
from jax.experimental import pallas as pl


def _add_body(x_ref, pe_ref, o_ref):
    i = pl.program_id(0)
    BLK = x_ref.shape[1]
    o_ref[...] = x_ref[...] + pe_ref[pl.ds(i * BLK, BLK), :][None, :, :]


def kernel(x, pe_table):
    B, S, D = x.shape
    BLK = 256
    n = S // BLK
    return pl.pallas_call(
        _add_body,
        out_shape=jax.ShapeDtypeStruct((B, S, D), x.dtype),
        grid=(n,),
        in_specs=[
            pl.BlockSpec((B, BLK, D), lambda i: (0, i, 0)),
            pl.BlockSpec((S, D), lambda i: (0, 0)),
        ],
        out_specs=pl.BlockSpec((B, BLK, D), lambda i: (0, i, 0)),
    )(x, pe_table)
